# baseline (speedup 1.0000x reference)
import functools

import jax
import jax.numpy as jnp
from jax import lax
from jax.experimental import pallas as pl
from jax.experimental.pallas import tpu as pltpu
from jax.experimental.pallas import tpu_sc as plsc

E = 320000
D = 128
N = 10000

NC = 2
NS = 16
H = N // NC
CHUNK = 128
TCHUNK = E // CHUNK
NBUF = 2
NP = 10240
NPT = NP // NS
ZROWS = 16
TRASH = N
WU = (H // NS) // 8 * 8


def _sc_segment_sum(x, index, params):
    mesh = plsc.VectorSubcoreMesh(core_axis_name="c", subcore_axis_name="s")
    scratch = (
        [pltpu.VMEM((CHUNK, D), jnp.float32) for _ in range(NBUF)]
        + [pltpu.VMEM((CHUNK,), jnp.int32) for _ in range(NBUF)]
        + [pltpu.VMEM((ZROWS, D), jnp.float32)]
        + [pltpu.VMEM((NC * NS, 16), jnp.int32)]
        + [pltpu.VMEM_SHARED((NP, D), jnp.float32)]
        + [pltpu.SemaphoreType.DMA for _ in range(2 * NBUF)]
    )

    @functools.partial(
        pl.kernel,
        out_type=jax.ShapeDtypeStruct((N, D), jnp.float32),
        mesh=mesh,
        scratch_types=scratch,
    )
    def k(x_hbm, idx_hbm, par_hbm, out_hbm, *refs):
        xbufs = refs[0:NBUF]
        ibufs = refs[NBUF:2 * NBUF]
        zbuf = refs[2 * NBUF]
        pbuf = refs[2 * NBUF + 1]
        acc = refs[2 * NBUF + 2]
        xsems = refs[2 * NBUF + 3:2 * NBUF + 3 + NBUF]
        isems = refs[2 * NBUF + 3 + NBUF:2 * NBUF + 3 + 2 * NBUF]

        cid = lax.axis_index("c")
        sid = lax.axis_index("s")

        pltpu.sync_copy(par_hbm, pbuf)
        pv = pbuf[cid * NS + sid]
        chunk_lo = pv[0]
        cnt = pv[1]
        nlo = cid * H
        nhi = nlo + H

        def start_load(c, b):
            base = (chunk_lo + c) * CHUNK
            pltpu.async_copy(x_hbm.at[pl.ds(base, CHUNK)], xbufs[b], xsems[b])
            pltpu.async_copy(idx_hbm.at[pl.ds(base, CHUNK)], ibufs[b], isems[b])

        def wait_load(b):
            pltpu.make_async_copy(x_hbm.at[pl.ds(0, CHUNK)], xbufs[b], xsems[b]).wait()
            pltpu.make_async_copy(idx_hbm.at[pl.ds(0, CHUNK)], ibufs[b], isems[b]).wait()

        for b in range(NBUF):
            @pl.when(b < cnt)
            def _():
                start_load(b, b)

        zero = jnp.zeros((16,), jnp.float32)

        def zrow(i, carry):
            for j in range(D // 16):
                zbuf[i, pl.ds(j * 16, 16)] = zero
            return carry

        lax.fori_loop(0, ZROWS, zrow, 0)
        for t in range(NPT // ZROWS):
            pltpu.sync_copy(zbuf, acc.at[pl.ds(sid * NPT + t * ZROWS, ZROWS)])
        plsc.subcore_barrier()

        def group(g, carry):
            for b in range(NBUF):
                c = g * NBUF + b

                @pl.when(c < cnt)
                def _():
                    wait_load(b)
                    for j in range(CHUNK // 16):
                        v = ibufs[b][pl.ds(j * 16, 16)]
                        keep = (v >= nlo) & (v < nhi)
                        ibufs[b][pl.ds(j * 16, 16)] = jnp.where(keep, v, TRASH)
                    pltpu.sync_copy(xbufs[b], acc.at[ibufs[b]], add=True)

                    @pl.when(c + NBUF < cnt)
                    def _():
                        start_load(c + NBUF, b)

            return carry

        lax.fori_loop(0, (cnt + NBUF - 1) // NBUF, group, 0)
        plsc.subcore_barrier()

        pltpu.sync_copy(
            acc.at[pl.ds(nlo + sid * WU, WU)],
            out_hbm.at[pl.ds(nlo + sid * WU, WU)],
        )
        rem = H - NS * WU

        @pl.when(sid == NS - 1)
        def _():
            pltpu.sync_copy(
                acc.at[pl.ds(nlo + NS * WU, rem)],
                out_hbm.at[pl.ds(nlo + NS * WU, rem)],
            )

    return k(x, index, params)


def kernel(x, index):
    s = jnp.sum((index < H).astype(jnp.int32))
    c0_end = (s + CHUNK - 1) // CHUNK
    c1_start = s // CHUNK
    t = jnp.arange(NS, dtype=jnp.int32)
    l0 = c0_end
    lo0 = t * l0 // NS
    cnt0 = (t + 1) * l0 // NS - lo0
    l1 = TCHUNK - c1_start
    lo1 = c1_start + t * l1 // NS
    cnt1 = c1_start + (t + 1) * l1 // NS - lo1
    lo = jnp.concatenate([lo0, lo1]).astype(jnp.int32)
    cnt = jnp.concatenate([cnt0, cnt1]).astype(jnp.int32)
    params = jnp.zeros((NC * NS, 16), jnp.int32)
    params = params.at[:, 0].set(lo).at[:, 1].set(cnt)
    return _sc_segment_sum(x, index, params)


# speedup vs baseline: 1.0730x; 1.0730x over previous
import functools

import jax
import jax.numpy as jnp
from jax import lax
from jax.experimental import pallas as pl
from jax.experimental.pallas import tpu as pltpu
from jax.experimental.pallas import tpu_sc as plsc

E = 320000
D = 128
N = 10000

NC = 2
NS = 16
H = N // NC
CHUNK = 64
TCHUNK = E // CHUNK
NBUF = 5
NP = 10240
NPT = NP // NS
ZROWS = 16
TRASH = N
WU = (H // NS) // 8 * 8


def _sc_segment_sum(x, index, params):
    mesh = plsc.VectorSubcoreMesh(core_axis_name="c", subcore_axis_name="s")
    scratch = (
        [pltpu.VMEM((CHUNK, D), jnp.float32) for _ in range(NBUF)]
        + [pltpu.VMEM((CHUNK,), jnp.int32) for _ in range(NBUF)]
        + [pltpu.VMEM((ZROWS, D), jnp.float32)]
        + [pltpu.VMEM((NC * NS, 16), jnp.int32)]
        + [pltpu.VMEM_SHARED((NP, D), jnp.float32)]
        + [pltpu.SemaphoreType.DMA for _ in range(2 * NBUF)]
    )

    @functools.partial(
        pl.kernel,
        out_type=jax.ShapeDtypeStruct((N, D), jnp.float32),
        mesh=mesh,
        scratch_types=scratch,
    )
    def k(x_hbm, idx_hbm, par_hbm, out_hbm, *refs):
        xbufs = refs[0:NBUF]
        ibufs = refs[NBUF:2 * NBUF]
        zbuf = refs[2 * NBUF]
        pbuf = refs[2 * NBUF + 1]
        acc = refs[2 * NBUF + 2]
        xsems = refs[2 * NBUF + 3:2 * NBUF + 3 + NBUF]
        isems = refs[2 * NBUF + 3 + NBUF:2 * NBUF + 3 + 2 * NBUF]

        cid = lax.axis_index("c")
        sid = lax.axis_index("s")

        pltpu.sync_copy(par_hbm, pbuf)
        pv = pbuf[cid * NS + sid]
        chunk_lo = pv[0]
        cnt = pv[1]
        nlo = cid * H
        nhi = nlo + H

        def start_load(c, b):
            base = (chunk_lo + c) * CHUNK
            pltpu.async_copy(x_hbm.at[pl.ds(base, CHUNK)], xbufs[b], xsems[b])
            pltpu.async_copy(idx_hbm.at[pl.ds(base, CHUNK)], ibufs[b], isems[b])

        def wait_load(b):
            pltpu.make_async_copy(x_hbm.at[pl.ds(0, CHUNK)], xbufs[b], xsems[b]).wait()
            pltpu.make_async_copy(idx_hbm.at[pl.ds(0, CHUNK)], ibufs[b], isems[b]).wait()

        for b in range(NBUF):
            @pl.when(b < cnt)
            def _():
                start_load(b, b)

        zero = jnp.zeros((16,), jnp.float32)

        def zrow(i, carry):
            for j in range(D // 16):
                zbuf[i, pl.ds(j * 16, 16)] = zero
            return carry

        lax.fori_loop(0, ZROWS, zrow, 0)
        for t in range(NPT // ZROWS):
            pltpu.sync_copy(zbuf, acc.at[pl.ds(sid * NPT + t * ZROWS, ZROWS)])
        plsc.subcore_barrier()

        def group(g, carry):
            for b in range(NBUF):
                c = g * NBUF + b

                @pl.when(c < cnt)
                def _():
                    wait_load(b)
                    for j in range(CHUNK // 16):
                        v = ibufs[b][pl.ds(j * 16, 16)]
                        keep = (v >= nlo) & (v < nhi)
                        ibufs[b][pl.ds(j * 16, 16)] = jnp.where(keep, v, TRASH)
                    pltpu.sync_copy(xbufs[b], acc.at[ibufs[b]], add=True)

                    @pl.when(c + NBUF < cnt)
                    def _():
                        start_load(c + NBUF, b)

            return carry

        lax.fori_loop(0, (cnt + NBUF - 1) // NBUF, group, 0)
        plsc.subcore_barrier()

        pltpu.sync_copy(
            acc.at[pl.ds(nlo + sid * WU, WU)],
            out_hbm.at[pl.ds(nlo + sid * WU, WU)],
        )
        rem = H - NS * WU

        @pl.when(sid == NS - 1)
        def _():
            pltpu.sync_copy(
                acc.at[pl.ds(nlo + NS * WU, rem)],
                out_hbm.at[pl.ds(nlo + NS * WU, rem)],
            )

    return k(x, index, params)


def kernel(x, index):
    s = jnp.sum((index < H).astype(jnp.int32))
    c0_end = (s + CHUNK - 1) // CHUNK
    c1_start = s // CHUNK
    t = jnp.arange(NS, dtype=jnp.int32)
    l0 = c0_end
    lo0 = t * l0 // NS
    cnt0 = (t + 1) * l0 // NS - lo0
    l1 = TCHUNK - c1_start
    lo1 = c1_start + t * l1 // NS
    cnt1 = c1_start + (t + 1) * l1 // NS - lo1
    lo = jnp.concatenate([lo0, lo1]).astype(jnp.int32)
    cnt = jnp.concatenate([cnt0, cnt1]).astype(jnp.int32)
    params = jnp.zeros((NC * NS, 16), jnp.int32)
    params = params.at[:, 0].set(lo).at[:, 1].set(cnt)
    return _sc_segment_sum(x, index, params)


# speedup vs baseline: 1.0871x; 1.0131x over previous
import functools

import jax
import jax.numpy as jnp
from jax import lax
from jax.experimental import pallas as pl
from jax.experimental.pallas import tpu as pltpu
from jax.experimental.pallas import tpu_sc as plsc

E = 320000
D = 128
N = 10000

NC = 2
NS = 16
H = N // NC
CHUNK = 80
TCHUNK = E // CHUNK
NBUF = 4
NP = 10240
NPT = NP // NS
ZROWS = 16
TRASH = N
WU = (H // NS) // 8 * 8


def _sc_segment_sum(x, index, params):
    mesh = plsc.VectorSubcoreMesh(core_axis_name="c", subcore_axis_name="s")
    scratch = (
        [pltpu.VMEM((CHUNK, D), jnp.float32) for _ in range(NBUF)]
        + [pltpu.VMEM((CHUNK,), jnp.int32) for _ in range(NBUF)]
        + [pltpu.VMEM((ZROWS, D), jnp.float32)]
        + [pltpu.VMEM((NC * NS, 16), jnp.int32)]
        + [pltpu.VMEM_SHARED((NP, D), jnp.float32)]
        + [pltpu.SemaphoreType.DMA for _ in range(2 * NBUF)]
    )

    @functools.partial(
        pl.kernel,
        out_type=jax.ShapeDtypeStruct((N, D), jnp.float32),
        mesh=mesh,
        scratch_types=scratch,
    )
    def k(x_hbm, idx_hbm, par_hbm, out_hbm, *refs):
        xbufs = refs[0:NBUF]
        ibufs = refs[NBUF:2 * NBUF]
        zbuf = refs[2 * NBUF]
        pbuf = refs[2 * NBUF + 1]
        acc = refs[2 * NBUF + 2]
        xsems = refs[2 * NBUF + 3:2 * NBUF + 3 + NBUF]
        isems = refs[2 * NBUF + 3 + NBUF:2 * NBUF + 3 + 2 * NBUF]

        cid = lax.axis_index("c")
        sid = lax.axis_index("s")

        pltpu.sync_copy(par_hbm, pbuf)
        pv = pbuf[cid * NS + sid]
        chunk_lo = pv[0]
        cnt = pv[1]
        nlo = cid * H
        nhi = nlo + H

        def start_load(c, b):
            base = (chunk_lo + c) * CHUNK
            pltpu.async_copy(x_hbm.at[pl.ds(base, CHUNK)], xbufs[b], xsems[b])
            pltpu.async_copy(idx_hbm.at[pl.ds(base, CHUNK)], ibufs[b], isems[b])

        def wait_load(b):
            pltpu.make_async_copy(x_hbm.at[pl.ds(0, CHUNK)], xbufs[b], xsems[b]).wait()
            pltpu.make_async_copy(idx_hbm.at[pl.ds(0, CHUNK)], ibufs[b], isems[b]).wait()

        for b in range(NBUF):
            @pl.when(b < cnt)
            def _():
                start_load(b, b)

        zero = jnp.zeros((16,), jnp.float32)

        def zrow(i, carry):
            for j in range(D // 16):
                zbuf[i, pl.ds(j * 16, 16)] = zero
            return carry

        lax.fori_loop(0, ZROWS, zrow, 0)
        for t in range(NPT // ZROWS):
            pltpu.sync_copy(zbuf, acc.at[pl.ds(sid * NPT + t * ZROWS, ZROWS)])
        plsc.subcore_barrier()

        def group(g, carry):
            for b in range(NBUF):
                c = g * NBUF + b

                @pl.when(c < cnt)
                def _():
                    wait_load(b)
                    for j in range(CHUNK // 16):
                        v = ibufs[b][pl.ds(j * 16, 16)]
                        keep = (v >= nlo) & (v < nhi)
                        ibufs[b][pl.ds(j * 16, 16)] = jnp.where(keep, v, TRASH)
                    pltpu.sync_copy(xbufs[b], acc.at[ibufs[b]], add=True)

                    @pl.when(c + NBUF < cnt)
                    def _():
                        start_load(c + NBUF, b)

            return carry

        lax.fori_loop(0, (cnt + NBUF - 1) // NBUF, group, 0)
        plsc.subcore_barrier()

        pltpu.sync_copy(
            acc.at[pl.ds(nlo + sid * WU, WU)],
            out_hbm.at[pl.ds(nlo + sid * WU, WU)],
        )
        rem = H - NS * WU

        @pl.when(sid == NS - 1)
        def _():
            pltpu.sync_copy(
                acc.at[pl.ds(nlo + NS * WU, rem)],
                out_hbm.at[pl.ds(nlo + NS * WU, rem)],
            )

    return k(x, index, params)


def kernel(x, index):
    s = jnp.sum((index < H).astype(jnp.int32))
    c0_end = (s + CHUNK - 1) // CHUNK
    c1_start = s // CHUNK
    t = jnp.arange(NS, dtype=jnp.int32)
    l0 = c0_end
    lo0 = t * l0 // NS
    cnt0 = (t + 1) * l0 // NS - lo0
    l1 = TCHUNK - c1_start
    lo1 = c1_start + t * l1 // NS
    cnt1 = c1_start + (t + 1) * l1 // NS - lo1
    lo = jnp.concatenate([lo0, lo1]).astype(jnp.int32)
    cnt = jnp.concatenate([cnt0, cnt1]).astype(jnp.int32)
    params = jnp.zeros((NC * NS, 16), jnp.int32)
    params = params.at[:, 0].set(lo).at[:, 1].set(cnt)
    return _sc_segment_sum(x, index, params)
